# Initial kernel scaffold; baseline (speedup 1.0000x reference)
import jax
import jax.numpy as jnp
from jax.experimental import pallas as pl


def kernel(x, table, gamma, beta):
    raise NotImplementedError("write your pallas kernel here")


# speedup vs baseline: 1.9076x; 1.9076x over previous
import functools

import jax
import jax.numpy as jnp
from jax import lax
from jax.experimental import pallas as pl
from jax.experimental.pallas import tpu as pltpu
from jax.experimental.pallas import tpu_sc as plsc

D = 128
L = 16
B_TOTAL = 4096 * 50
CHUNK = 128


@functools.cache
def _build():
    info = plsc.get_sparse_core_info()
    NC, NS = info.num_cores, info.num_subcores
    NW = NC * NS
    rows_per_w = B_TOTAL // NW
    n_chunks = rows_per_w // CHUNK
    mesh = plsc.VectorSubcoreMesh(core_axis_name="c", subcore_axis_name="s")

    @functools.partial(
        pl.kernel,
        mesh=mesh,
        out_type=jax.ShapeDtypeStruct((B_TOTAL, D), jnp.float32),
        scratch_types=[
            pltpu.VMEM((n_chunks, CHUNK), jnp.int32),
            pltpu.VMEM((CHUNK, D), jnp.float32),
            pltpu.VMEM((D,), jnp.float32),
            pltpu.VMEM((D,), jnp.float32),
            pltpu.SemaphoreType.DMA,
        ],
    )
    def k(x_hbm, table_hbm, gamma_hbm, beta_hbm, out_hbm,
          idx_v, rows_v, gamma_v, beta_v, sem):
        wid = lax.axis_index("s") * NC + lax.axis_index("c")
        pltpu.sync_copy(x_hbm.at[wid], idx_v)
        pltpu.sync_copy(gamma_hbm, gamma_v)
        pltpu.sync_copy(beta_hbm, beta_v)
        gam = [gamma_v[pl.ds(j * L, L)] for j in range(D // L)]
        bet = [beta_v[pl.ds(j * L, L)] for j in range(D // L)]
        lanes = jnp.arange(L, dtype=jnp.int32)
        shuffles = [lanes ^ m for m in (8, 4, 2, 1)]

        dnums = lax.GatherDimensionNumbers(
            offset_dims=(), collapsed_slice_dims=(0,), start_index_map=(0,))

        def lane_total(v):
            for s in shuffles:
                v = v + lax.gather(
                    v, s[:, None], dnums, slice_sizes=(1,),
                    mode=lax.GatherScatterMode.PROMISE_IN_BOUNDS)
            return v

        def chunk_body(c, carry):
            pltpu.async_copy(table_hbm.at[idx_v.at[c]], rows_v, sem).wait()

            def row_body(r, rcarry):
                parts = [rows_v[r, pl.ds(j * L, L)] for j in range(D // L)]
                s = parts[0]
                sq = parts[0] * parts[0]
                for p in parts[1:]:
                    s = s + p
                    sq = sq + p * p
                total = lane_total(s)
                totsq = lane_total(sq)
                mean = total * (1.0 / D)
                var = totsq * (1.0 / D) - mean * mean
                vpe = var + 1e-5
                seed = jnp.int32(0x5F3759DF) - (
                    lax.bitcast_convert_type(vpe, jnp.int32) >> 1)
                y = lax.bitcast_convert_type(seed, jnp.float32)
                y = y * (1.5 - 0.5 * vpe * y * y)
                y = y * (1.5 - 0.5 * vpe * y * y)
                y = y * (1.5 - 0.5 * vpe * y * y)
                for j in range(D // L):
                    t = (parts[j] - mean) * y
                    rows_v[r, pl.ds(j * L, L)] = t * gam[j] + bet[j]
                return rcarry

            lax.fori_loop(0, CHUNK, row_body, 0)
            pltpu.sync_copy(
                rows_v, out_hbm.at[pl.ds(wid * rows_per_w + c * CHUNK, CHUNK)])
            return carry

        lax.fori_loop(0, n_chunks, chunk_body, 0)

    def run(x, table, gamma, beta):
        xf = x.astype(jnp.int32).reshape(NW, n_chunks, CHUNK)
        out = k(xf, table, gamma, beta)
        return out.reshape(x.shape[0], x.shape[1], D)

    return run


def kernel(x, table, gamma, beta):
    return _build()(x, table, gamma, beta)


# speedup vs baseline: 2.2078x; 1.1574x over previous
import functools

import jax
import jax.numpy as jnp
from jax import lax
from jax.experimental import pallas as pl
from jax.experimental.pallas import tpu as pltpu
from jax.experimental.pallas import tpu_sc as plsc

D = 128
L = 16
B_TOTAL = 4096 * 50
CHUNK = 128
UNROLL = 4


@functools.cache
def _build():
    info = plsc.get_sparse_core_info()
    NC, NS = info.num_cores, info.num_subcores
    NW = NC * NS
    rows_per_w = B_TOTAL // NW
    n_chunks = rows_per_w // CHUNK
    mesh = plsc.VectorSubcoreMesh(core_axis_name="c", subcore_axis_name="s")

    @functools.partial(
        pl.kernel,
        mesh=mesh,
        out_type=jax.ShapeDtypeStruct((B_TOTAL, D), jnp.float32),
        scratch_types=[
            pltpu.VMEM((n_chunks, CHUNK), jnp.int32),
            pltpu.VMEM((4, CHUNK, D), jnp.float32),
            pltpu.VMEM((D,), jnp.float32),
            pltpu.VMEM((D,), jnp.float32),
            pltpu.SemaphoreType.DMA,
            pltpu.SemaphoreType.DMA,
        ],
    )
    def k(x_hbm, table_hbm, gamma_hbm, beta_hbm, out_hbm,
          idx_v, rows_v, gamma_v, beta_v, gsem, osem):
        wid = lax.axis_index("s") * NC + lax.axis_index("c")
        out_base = wid * rows_per_w
        pltpu.sync_copy(x_hbm.at[wid], idx_v)
        pltpu.sync_copy(gamma_hbm, gamma_v)
        pltpu.sync_copy(beta_hbm, beta_v)
        gam = [gamma_v[pl.ds(j * L, L)] for j in range(D // L)]
        bet = [beta_v[pl.ds(j * L, L)] for j in range(D // L)]
        lanes = jnp.arange(L, dtype=jnp.int32)
        shuffles = [lanes ^ m for m in (8, 4, 2, 1)]
        dnums = lax.GatherDimensionNumbers(
            offset_dims=(), collapsed_slice_dims=(0,), start_index_map=(0,))

        def lane_total(v):
            for s in shuffles:
                v = v + lax.gather(
                    v, s[:, None], dnums, slice_sizes=(1,),
                    mode=lax.GatherScatterMode.PROMISE_IN_BOUNDS)
            return v

        def gather_start(c):
            pltpu.async_copy(table_hbm.at[idx_v.at[c]], rows_v.at[c & 3], gsem)

        def gather_wait(c):
            pltpu.make_async_copy(
                table_hbm.at[idx_v.at[c]], rows_v.at[c & 3], gsem).wait()

        def out_start(c):
            pltpu.async_copy(
                rows_v.at[c & 3],
                out_hbm.at[pl.ds(out_base + c * CHUNK, CHUNK)], osem)

        def out_wait(c):
            pltpu.make_async_copy(
                rows_v.at[c & 3],
                out_hbm.at[pl.ds(out_base + c * CHUNK, CHUNK)], osem).wait()

        def normalize_row(buf, r):
            parts = [buf[r, pl.ds(j * L, L)] for j in range(D // L)]
            s = parts[0]
            sq = parts[0] * parts[0]
            for p in parts[1:]:
                s = s + p
                sq = sq + p * p
            total = lane_total(s)
            totsq = lane_total(sq)
            mean = total * (1.0 / D)
            var = totsq * (1.0 / D) - mean * mean
            vpe = var + 1e-5
            seed = jnp.int32(0x5F3759DF) - (
                lax.bitcast_convert_type(vpe, jnp.int32) >> 1)
            y = lax.bitcast_convert_type(seed, jnp.float32)
            y = y * (1.5 - 0.5 * vpe * y * y)
            y = y * (1.5 - 0.5 * vpe * y * y)
            y = y * (1.5 - 0.5 * vpe * y * y)
            for j in range(D // L):
                t = (parts[j] - mean) * y
                buf[r, pl.ds(j * L, L)] = t * gam[j] + bet[j]

        def chunk_body(c, carry):
            @pl.when(c >= 3)
            def _():
                out_wait(c - 3)

            @pl.when(c + 1 < n_chunks)
            def _():
                gather_start(c + 1)

            gather_wait(c)
            buf = rows_v.at[c & 3]

            def row_body(g, rcarry):
                for u in range(UNROLL):
                    normalize_row(buf, g * UNROLL + u)
                return rcarry

            lax.fori_loop(0, CHUNK // UNROLL, row_body, 0)
            out_start(c)
            return carry

        gather_start(0)
        lax.fori_loop(0, n_chunks, chunk_body, 0)
        out_wait(n_chunks - 3)
        out_wait(n_chunks - 2)
        out_wait(n_chunks - 1)

    def run(x, table, gamma, beta):
        xf = x.astype(jnp.int32).reshape(NW, n_chunks, CHUNK)
        out = k(xf, table, gamma, beta)
        return out.reshape(x.shape[0], x.shape[1], D)

    return run


def kernel(x, table, gamma, beta):
    return _build()(x, table, gamma, beta)


# speedup vs baseline: 3.3242x; 1.5057x over previous
import functools

import jax
import jax.numpy as jnp
from jax import lax
from jax.experimental import pallas as pl
from jax.experimental.pallas import tpu as pltpu
from jax.experimental.pallas import tpu_sc as plsc

D = 128
L = 16
BATCH = 4096
HIST = 50
CHUNK_B = 2
CHUNK = CHUNK_B * HIST
UNROLL = 4


@functools.cache
def _build():
    info = plsc.get_sparse_core_info()
    NC, NS = info.num_cores, info.num_subcores
    NW = NC * NS
    b_per_w = BATCH // NW
    n_chunks = b_per_w // CHUNK_B
    mesh = plsc.VectorSubcoreMesh(core_axis_name="c", subcore_axis_name="s")

    @functools.partial(
        pl.kernel,
        mesh=mesh,
        out_type=jax.ShapeDtypeStruct((BATCH, HIST, D), jnp.float32),
        scratch_types=[
            pltpu.VMEM((n_chunks, CHUNK), jnp.int32),
            pltpu.VMEM((4, CHUNK, D), jnp.float32),
            pltpu.VMEM((D,), jnp.float32),
            pltpu.VMEM((D,), jnp.float32),
            pltpu.SemaphoreType.DMA,
            pltpu.SemaphoreType.DMA,
        ],
    )
    def k(x_hbm, table_hbm, gamma_hbm, beta_hbm, out_hbm,
          idx_v, rows_v, gamma_v, beta_v, gsem, osem):
        wid = lax.axis_index("s") * NC + lax.axis_index("c")
        b_base = wid * b_per_w
        pltpu.sync_copy(x_hbm.at[wid], idx_v)
        pltpu.sync_copy(gamma_hbm, gamma_v)
        pltpu.sync_copy(beta_hbm, beta_v)
        gam = [gamma_v[pl.ds(j * L, L)] for j in range(D // L)]
        bet = [beta_v[pl.ds(j * L, L)] for j in range(D // L)]
        lanes = jnp.arange(L, dtype=jnp.int32)
        shuffles = [lanes ^ m for m in (8, 4, 2, 1)]
        dnums = lax.GatherDimensionNumbers(
            offset_dims=(), collapsed_slice_dims=(0,), start_index_map=(0,))

        def lane_total(v):
            for s in shuffles:
                v = v + lax.gather(
                    v, s[:, None], dnums, slice_sizes=(1,),
                    mode=lax.GatherScatterMode.PROMISE_IN_BOUNDS)
            return v

        def gather_start(c):
            pltpu.async_copy(table_hbm.at[idx_v.at[c]], rows_v.at[c & 3], gsem)

        def gather_wait(c):
            pltpu.make_async_copy(
                table_hbm.at[idx_v.at[c]], rows_v.at[c & 3], gsem).wait()

        def out_start(c):
            buf = rows_v.at[c & 3]
            b = b_base + c * CHUNK_B
            for i in range(CHUNK_B):
                pltpu.async_copy(
                    buf.at[pl.ds(i * HIST, HIST)], out_hbm.at[b + i], osem)

        def out_wait(c):
            buf = rows_v.at[c & 3]
            b = b_base + c * CHUNK_B
            for i in range(CHUNK_B):
                pltpu.make_async_copy(
                    buf.at[pl.ds(i * HIST, HIST)], out_hbm.at[b + i],
                    osem).wait()

        def normalize_row(buf, r):
            parts = [buf[r, pl.ds(j * L, L)] for j in range(D // L)]
            sqs = [p * p for p in parts]
            while len(parts) > 1:
                parts = [parts[i] + parts[i + 1]
                         for i in range(0, len(parts), 2)] + parts[len(parts) & ~1:]
            while len(sqs) > 1:
                sqs = [sqs[i] + sqs[i + 1]
                       for i in range(0, len(sqs), 2)] + sqs[len(sqs) & ~1:]
            total = lane_total(parts[0])
            totsq = lane_total(sqs[0])
            mean = total * (1.0 / D)
            var = totsq * (1.0 / D) - mean * mean
            vpe = var + 1e-5
            seed = jnp.int32(0x5F3759DF) - (
                lax.bitcast_convert_type(vpe, jnp.int32) >> 1)
            y = lax.bitcast_convert_type(seed, jnp.float32)
            y = y * (1.5 - 0.5 * vpe * y * y)
            y = y * (1.5 - 0.5 * vpe * y * y)
            for j in range(D // L):
                t = (buf[r, pl.ds(j * L, L)] - mean) * y
                buf[r, pl.ds(j * L, L)] = t * gam[j] + bet[j]

        def chunk_body(c, carry):
            @pl.when(c >= 3)
            def _():
                out_wait(c - 3)

            @pl.when(c + 1 < n_chunks)
            def _():
                gather_start(c + 1)

            gather_wait(c)
            buf = rows_v.at[c & 3]

            def row_body(g, rcarry):
                for u in range(UNROLL):
                    normalize_row(buf, g * UNROLL + u)
                return rcarry

            lax.fori_loop(0, CHUNK // UNROLL, row_body, 0)
            out_start(c)
            return carry

        gather_start(0)
        lax.fori_loop(0, n_chunks, chunk_body, 0)
        out_wait(n_chunks - 3)
        out_wait(n_chunks - 2)
        out_wait(n_chunks - 1)

    def run(x, table, gamma, beta):
        xf = x.astype(jnp.int32).reshape(NW, n_chunks, CHUNK)
        return k(xf, table, gamma, beta)

    return run


def kernel(x, table, gamma, beta):
    return _build()(x, table, gamma, beta)
